# baseline (speedup 1.0000x reference)
import jax
import jax.numpy as jnp
from jax.experimental import pallas as pl
from jax.experimental.pallas import tpu as pltpu

_B, _D, _N = 16, 256, 1024
_NE = 1024
_NT = 512


def _vq_body(x_ref, e_ref, et_ref, revj_ref, q_ref, idx_ref, loss_ref):
    b = pl.program_id(0)
    k = pl.program_id(1)
    x = x_ref[0]
    e = e_ref[...]
    et = et_ref[...]
    revj = revj_ref[...]

    scores2 = jax.lax.dot_general(
        et, x, (((1,), (0,)), ((), ())),
        preferred_element_type=jnp.float32)
    e_sq = jnp.sum(e * e, axis=0, keepdims=True).reshape(_NE, 1)
    x_sq = jnp.sum(x * x, axis=0, keepdims=True)
    dist = (x_sq + e_sq) - scores2

    minval = jnp.min(dist, axis=0, keepdims=True)
    masked = jnp.where(dist == minval, revj, 0.0)
    maxrev = jnp.max(masked, axis=0, keepdims=True)
    idx_ref[0] = (jnp.float32(_NE) - maxrev).astype(jnp.int32)
    h = jnp.where(revj == maxrev, 1.0, 0.0)
    q_ref[0] = jax.lax.dot_general(
        e, h, (((1,), (0,)), ((), ())),
        preferred_element_type=jnp.float32)

    part = jnp.sum(minval)
    first = jnp.logical_and(b == 0, k == 0)

    @pl.when(first)
    def _():
        loss_ref[0, 0] = part

    @pl.when(jnp.logical_not(first))
    def _():
        loss_ref[0, 0] = loss_ref[0, 0] + part


def kernel(inputs, embedding):
    emb_t = 2.0 * embedding.T
    revj = (jnp.float32(_NE)
            - jnp.arange(_NE, dtype=jnp.float32)).reshape(_NE, 1)

    nk = _N // _NT
    q, idx, loss_sum = pl.pallas_call(
        _vq_body,
        grid=(_B, nk),
        in_specs=[
            pl.BlockSpec((1, _D, _NT), lambda b, k: (b, 0, k)),
            pl.BlockSpec((_D, _NE), lambda b, k: (0, 0)),
            pl.BlockSpec((_NE, _D), lambda b, k: (0, 0)),
            pl.BlockSpec((_NE, 1), lambda b, k: (0, 0)),
        ],
        out_specs=[
            pl.BlockSpec((1, _D, _NT), lambda b, k: (b, 0, k)),
            pl.BlockSpec((1, 1, _NT), lambda b, k: (b, 0, k)),
            pl.BlockSpec((1, 1), lambda b, k: (0, 0),
                         memory_space=pltpu.SMEM),
        ],
        out_shape=[
            jax.ShapeDtypeStruct((_B, _D, _N), jnp.float32),
            jax.ShapeDtypeStruct((_B, 1, _N), jnp.int32),
            jax.ShapeDtypeStruct((1, 1), jnp.float32),
        ],
    )(inputs, embedding, emb_t, revj)

    loss = loss_sum[0, 0] / jnp.float32(_B * _D * _N)
    return (q, idx.reshape(_B, _N), loss)


# speedup vs baseline: 1.2426x; 1.2426x over previous
import jax
import jax.numpy as jnp
from jax.experimental import pallas as pl
from jax.experimental.pallas import tpu as pltpu

_B, _D, _N = 16, 256, 1024
_NE = 1024


def _vq_body(x_ref, e_ref, et_ref, revj_ref, q_ref, idx_ref, loss_ref):
    b = pl.program_id(0)
    x = x_ref[0]
    e = e_ref[...]
    et = et_ref[...]
    revj = revj_ref[...]

    scores2 = jax.lax.dot_general(
        et, x, (((1,), (0,)), ((), ())),
        preferred_element_type=jnp.float32)
    e_sq = jnp.sum(e * e, axis=0, keepdims=True).reshape(_NE, 1)
    x_sq = jnp.sum(x * x, axis=0, keepdims=True)
    dist = (x_sq + e_sq) - scores2

    minval = jnp.min(dist, axis=0, keepdims=True)
    masked = jnp.where(dist == minval, revj, 0.0)
    maxrev = jnp.max(masked, axis=0, keepdims=True)
    idx_ref[0] = (jnp.float32(_NE) - maxrev).astype(jnp.int32)
    h = jnp.where(revj == maxrev, 1.0, 0.0)
    q_ref[0] = jax.lax.dot_general(
        e, h, (((1,), (0,)), ((), ())),
        preferred_element_type=jnp.float32)

    part = jnp.sum(minval)

    @pl.when(b == 0)
    def _():
        loss_ref[0, 0] = part

    @pl.when(b > 0)
    def _():
        loss_ref[0, 0] = loss_ref[0, 0] + part


def kernel(inputs, embedding):
    emb_t = 2.0 * embedding.T
    revj = (jnp.float32(_NE)
            - jnp.arange(_NE, dtype=jnp.float32)).reshape(_NE, 1)

    q, idx, loss_sum = pl.pallas_call(
        _vq_body,
        grid=(_B,),
        in_specs=[
            pl.BlockSpec((1, _D, _N), lambda b: (b, 0, 0)),
            pl.BlockSpec((_D, _NE), lambda b: (0, 0)),
            pl.BlockSpec((_NE, _D), lambda b: (0, 0)),
            pl.BlockSpec((_NE, 1), lambda b: (0, 0)),
        ],
        out_specs=[
            pl.BlockSpec((1, _D, _N), lambda b: (b, 0, 0)),
            pl.BlockSpec((1, 1, _N), lambda b: (b, 0, 0)),
            pl.BlockSpec((1, 1), lambda b: (0, 0), memory_space=pltpu.SMEM),
        ],
        out_shape=[
            jax.ShapeDtypeStruct((_B, _D, _N), jnp.float32),
            jax.ShapeDtypeStruct((_B, 1, _N), jnp.int32),
            jax.ShapeDtypeStruct((1, 1), jnp.float32),
        ],
    )(inputs, embedding, emb_t, revj)

    loss = loss_sum[0, 0] / jnp.float32(_B * _D * _N)
    return (q, idx.reshape(_B, _N), loss)
